# baseline (speedup 1.0000x reference)
import functools

import jax
import jax.numpy as jnp
from jax import lax
from jax.experimental import pallas as pl
from jax.experimental.pallas import tpu as pltpu
from jax.experimental.pallas import tpu_sc as plsc

_N, _E, _D = 10000, 320000, 128
_NC, _NS = 2, 16
_NW = _NC * _NS
_CH = 128
_CPW = 80
_EPW = _CH * _CPW
_E_PAD = _NW * _EPW
_N_PAD = 10240
_RPS = _N_PAD // _NS
_NBUF = 4
_LOOK = 2


def _sc_gather_segsum(x, eidx):
    mesh = plsc.VectorSubcoreMesh(
        core_axis_name="c", subcore_axis_name="s",
        num_cores=_NC, num_subcores=_NS)

    @functools.partial(
        pl.kernel,
        out_type=(jax.ShapeDtypeStruct((_NC, _N_PAD, _D), jnp.float32),
                  jax.ShapeDtypeStruct((_NC, _N_PAD), jnp.float32)),
        mesh=mesh,
        scratch_types=(
            [
                pltpu.VMEM_SHARED((_N_PAD, _D), jnp.float32),
                pltpu.VMEM_SHARED((_N_PAD,), jnp.float32),
                pltpu.VMEM((_CH,), jnp.float32),
            ]
            + [pltpu.VMEM((_CH, _D), jnp.float32)] * 2
            + [pltpu.VMEM((2, _CH), jnp.int32)] * 4
            + [pltpu.SemaphoreType.DMA] * 2
            + [pltpu.SemaphoreType.DMA] * 2
            + [pltpu.SemaphoreType.DMA] * 2
            + [pltpu.SemaphoreType.DMA] * 4
        ))
    def k(x_hbm, e_hbm, acc_out, deg_out, acc_sh, deg_sh, ones_v, *bufs):
        rows = list(bufs[0:2])
        ei = list(bufs[2:6])
        semg = list(bufs[6:8])
        sems = list(bufs[8:10])
        semd = list(bufs[10:12])
        semi = list(bufs[12:16])

        cid = lax.axis_index("c")
        sid = lax.axis_index("s")
        wid = sid * _NC + cid
        cbase = wid * _CPW

        zero16 = jnp.zeros((16,), jnp.float32)
        one16 = jnp.ones((16,), jnp.float32)

        pltpu.async_copy(e_hbm.at[cbase], ei[0], semi[0])
        pltpu.async_copy(e_hbm.at[cbase + 1], ei[1], semi[1])

        def fill_ones(i, carry):
            ones_v[pl.ds(i * 16, 16)] = one16
            return carry
        lax.fori_loop(0, _CH // 16, fill_ones, 0)

        def fill_zero(i, carry):
            rows[0][i // (_D // 16), pl.ds((i % (_D // 16)) * 16, 16)] = zero16
            return carry
        lax.fori_loop(0, _CH * (_D // 16), fill_zero, 0)

        rbase = sid * _RPS

        def zero_acc(j, carry):
            pltpu.sync_copy(rows[0], acc_sh.at[pl.ds(rbase + j * _CH, _CH)])
            pltpu.sync_copy(rows[0].at[0],
                            deg_sh.at[pl.ds(rbase + j * _CH, _CH)])
            return carry
        lax.fori_loop(0, _RPS // _CH, zero_acc, 0)

        plsc.subcore_barrier()

        def wait_rows(sem, buf):
            pltpu.make_async_copy(x_hbm.at[pl.ds(0, _CH)], buf, sem).wait()

        def wait_deg(sem):
            pltpu.make_async_copy(x_hbm.at[0], ones_v, sem).wait()

        def wait_idx(q):
            pltpu.make_async_copy(e_hbm.at[0], ei[q], semi[q]).wait()

        wait_idx(0)
        pltpu.async_copy(x_hbm.at[ei[0].at[0]], rows[0], semg[0])

        def outer(i, carry):
            for b4 in range(4):
                c = i * 4 + b4
                rb = b4 % 2
                ob = 1 - rb
                q = b4
                q1 = (b4 + 1) % 4
                q2 = (b4 + 2) % 4

                def prefetch_idx():
                    pltpu.async_copy(e_hbm.at[cbase + c + 2], ei[q2], semi[q2])
                if b4 < 2:
                    prefetch_idx()
                else:
                    pl.when(i < _CPW // 4 - 1)(prefetch_idx)

                def wait_prev_scatter():
                    wait_rows(sems[ob], rows[ob])
                    wait_deg(semd[ob])
                if b4 > 0:
                    wait_prev_scatter()
                else:
                    pl.when(i > 0)(wait_prev_scatter)

                def gather_next():
                    wait_idx(q1)
                    pltpu.async_copy(x_hbm.at[ei[q1].at[0]], rows[ob],
                                     semg[ob])
                if b4 < 3:
                    gather_next()
                else:
                    pl.when(i < _CPW // 4 - 1)(gather_next)

                wait_rows(semg[rb], rows[rb])
                pltpu.async_copy(rows[rb], acc_sh.at[ei[q].at[1]], sems[rb],
                                 add=True)
                pltpu.async_copy(ones_v, deg_sh.at[ei[q].at[1]], semd[rb],
                                 add=True)
            return carry
        lax.fori_loop(0, _CPW // 4, outer, 0)

        wait_rows(sems[(_CPW - 1) % 2], rows[(_CPW - 1) % 2])
        wait_deg(semd[(_CPW - 1) % 2])

        plsc.subcore_barrier()

        pltpu.sync_copy(acc_sh.at[pl.ds(rbase, _RPS)],
                        acc_out.at[cid, pl.ds(rbase, _RPS)])
        pltpu.sync_copy(deg_sh.at[pl.ds(rbase, _RPS)],
                        deg_out.at[cid, pl.ds(rbase, _RPS)])

    return k(x, eidx)


def _tc_combine(x_pad, parts_flat, degs_flat, W_self, W_neigh, b2):
    bn = 512
    g = _N_PAD // bn

    def body(x_ref, p0_ref, p1_ref, d0_ref, d1_ref, ws_ref, wn_ref, b_ref,
             o_ref):
        deg = jnp.maximum(d0_ref[...] + d1_ref[...], 1.0)
        h = (p0_ref[...] + p1_ref[...]) / deg[:, None]
        out = (jnp.dot(x_ref[...], ws_ref[...],
                       preferred_element_type=jnp.float32)
               + jnp.dot(h, wn_ref[...], preferred_element_type=jnp.float32)
               + b_ref[...])
        o_ref[...] = jnp.maximum(out, 0.0)

    return pl.pallas_call(
        body,
        grid=(g,),
        in_specs=[
            pl.BlockSpec((bn, _D), lambda i: (i, 0)),
            pl.BlockSpec((bn, _D), lambda i: (i, 0)),
            pl.BlockSpec((bn, _D), lambda i: (i + g, 0)),
            pl.BlockSpec((bn,), lambda i: (i,)),
            pl.BlockSpec((bn,), lambda i: (i + g,)),
            pl.BlockSpec((_D, _D), lambda i: (0, 0)),
            pl.BlockSpec((_D, _D), lambda i: (0, 0)),
            pl.BlockSpec((1, _D), lambda i: (0, 0)),
        ],
        out_specs=pl.BlockSpec((bn, _D), lambda i: (i, 0)),
        out_shape=jax.ShapeDtypeStruct((_N_PAD, _D), jnp.float32),
    )(x_pad, parts_flat, parts_flat, degs_flat, degs_flat,
      W_self, W_neigh, b2)


def kernel(x, edge_index, W_self, W_neigh, b):
    src = edge_index[0]
    dst = edge_index[1]
    npad = _E_PAD - _E
    src_p = jnp.concatenate([src, jnp.zeros((npad,), jnp.int32)])
    dst_p = jnp.concatenate([dst, jnp.full((npad,), _N, jnp.int32)])
    eidx = jnp.stack([src_p.reshape(-1, _CH), dst_p.reshape(-1, _CH)], axis=1)
    acc, deg = _sc_gather_segsum(x, eidx)
    x_pad = jnp.concatenate([x, jnp.zeros((_N_PAD - _N, _D), x.dtype)])
    out = _tc_combine(x_pad, acc.reshape(-1, _D), deg.reshape(-1),
                      W_self, W_neigh, b.reshape(1, _D))
    return out[:_N]


# speedup vs baseline: 2.6170x; 2.6170x over previous
import functools

import jax
import jax.numpy as jnp
from jax import lax
from jax.experimental import pallas as pl
from jax.experimental.pallas import tpu as pltpu
from jax.experimental.pallas import tpu_sc as plsc

_N, _E, _D = 10000, 320000, 128
_NC, _NS = 2, 16
_NW = _NC * _NS
_CH = 128
_CPW = 80
_EPW = _CH * _CPW
_E_PAD = _NW * _EPW
_N_PAD = 10240
_RPS = _N_PAD // _NS
_NBUF = 4
_LOOK = 2


def _sc_gather_segsum(x, eidx):
    mesh = plsc.VectorSubcoreMesh(
        core_axis_name="c", subcore_axis_name="s",
        num_cores=_NC, num_subcores=_NS)

    @functools.partial(
        pl.kernel,
        out_type=(jax.ShapeDtypeStruct((_NC, _N_PAD, _D), jnp.float32),
                  jax.ShapeDtypeStruct((_NC, _N_PAD), jnp.float32)),
        mesh=mesh,
        scratch_types=(
            [
                pltpu.VMEM_SHARED((_N_PAD, _D), jnp.float32),
                pltpu.VMEM_SHARED((_N_PAD,), jnp.float32),
                pltpu.VMEM((_CH,), jnp.float32),
            ]
            + [pltpu.VMEM((_CH, _D), jnp.float32)] * 2
            + [pltpu.VMEM((2, _CH), jnp.int32)] * 4
            + [pltpu.SemaphoreType.DMA] * 2
            + [pltpu.SemaphoreType.DMA] * 2
            + [pltpu.SemaphoreType.DMA] * 2
            + [pltpu.SemaphoreType.DMA] * 4
        ))
    def k(x_hbm, e_hbm, acc_out, deg_out, acc_sh, deg_sh, ones_v, *bufs):
        rows = list(bufs[0:2])
        ei = list(bufs[2:6])
        semg = list(bufs[6:8])
        sems = list(bufs[8:10])
        semd = list(bufs[10:12])
        semi = list(bufs[12:16])

        cid = lax.axis_index("c")
        sid = lax.axis_index("s")
        wid = sid * _NC + cid
        cbase = wid * _CPW

        zero16 = jnp.zeros((16,), jnp.float32)
        one16 = jnp.ones((16,), jnp.float32)

        pltpu.async_copy(e_hbm.at[cbase], ei[0], semi[0])
        pltpu.async_copy(e_hbm.at[cbase + 1], ei[1], semi[1])

        def fill_ones(i, carry):
            ones_v[pl.ds(i * 16, 16)] = one16
            return carry
        lax.fori_loop(0, _CH // 16, fill_ones, 0)

        def fill_zero(i, carry):
            rows[0][i // (_D // 16), pl.ds((i % (_D // 16)) * 16, 16)] = zero16
            return carry
        lax.fori_loop(0, _CH * (_D // 16), fill_zero, 0)

        rbase = sid * _RPS

        def zero_acc(j, carry):
            pltpu.sync_copy(rows[0], acc_sh.at[pl.ds(rbase + j * _CH, _CH)])
            pltpu.sync_copy(rows[0].at[0],
                            deg_sh.at[pl.ds(rbase + j * _CH, _CH)])
            return carry
        lax.fori_loop(0, _RPS // _CH, zero_acc, 0)

        plsc.subcore_barrier()

        def wait_rows(sem, buf):
            pltpu.make_async_copy(x_hbm.at[pl.ds(0, _CH)], buf, sem).wait()

        def wait_deg(sem):
            pltpu.make_async_copy(x_hbm.at[0], ones_v, sem).wait()

        def wait_idx(q):
            pltpu.make_async_copy(e_hbm.at[0], ei[q], semi[q]).wait()

        wait_idx(0)
        pltpu.async_copy(x_hbm.at[ei[0].at[0]], rows[0], semg[0])

        def outer(i, carry):
            for b4 in range(4):
                c = i * 4 + b4
                rb = b4 % 2
                ob = 1 - rb
                q = b4
                q1 = (b4 + 1) % 4
                q2 = (b4 + 2) % 4

                def prefetch_idx():
                    pltpu.async_copy(e_hbm.at[cbase + c + 2], ei[q2], semi[q2])
                if b4 < 2:
                    prefetch_idx()
                else:
                    pl.when(i < _CPW // 4 - 1)(prefetch_idx)

                def wait_prev_scatter():
                    wait_rows(sems[ob], rows[ob])
                    wait_deg(semd[ob])
                if b4 > 0:
                    wait_prev_scatter()
                else:
                    pl.when(i > 0)(wait_prev_scatter)

                def gather_next():
                    wait_idx(q1)
                    pltpu.async_copy(x_hbm.at[ei[q1].at[0]], rows[ob],
                                     semg[ob])
                if b4 < 3:
                    gather_next()
                else:
                    pl.when(i < _CPW // 4 - 1)(gather_next)

                wait_rows(semg[rb], rows[rb])
                pltpu.async_copy(rows[rb], acc_sh.at[ei[q].at[1]], sems[rb],
                                 add=True)
                pltpu.async_copy(ones_v, deg_sh.at[ei[q].at[1]], semd[rb],
                                 add=True)
            return carry
        lax.fori_loop(0, _CPW // 4, outer, 0)

        wait_rows(sems[(_CPW - 1) % 2], rows[(_CPW - 1) % 2])
        wait_deg(semd[(_CPW - 1) % 2])

        plsc.subcore_barrier()

        pltpu.sync_copy(acc_sh.at[pl.ds(rbase, _RPS)],
                        acc_out.at[cid, pl.ds(rbase, _RPS)])
        pltpu.sync_copy(deg_sh.at[pl.ds(rbase, _RPS)],
                        deg_out.at[cid, pl.ds(rbase, _RPS)])

    return k(x, eidx)


def _tc_combine(x_pad, parts_flat, degs_flat, W_self, W_neigh, b2):
    bn = 512
    g = _N_PAD // bn

    def body(x_ref, p0_ref, p1_ref, d0_ref, d1_ref, ws_ref, wn_ref, b_ref,
             o_ref):
        deg = jnp.maximum(d0_ref[...] + d1_ref[...], 1.0)
        h = (p0_ref[...] + p1_ref[...]) / deg[:, None]
        out = (jnp.dot(x_ref[...], ws_ref[...],
                       preferred_element_type=jnp.float32)
               + jnp.dot(h, wn_ref[...], preferred_element_type=jnp.float32)
               + b_ref[...])
        o_ref[...] = jnp.maximum(out, 0.0)

    return pl.pallas_call(
        body,
        grid=(g,),
        in_specs=[
            pl.BlockSpec((bn, _D), lambda i: (i, 0)),
            pl.BlockSpec((bn, _D), lambda i: (i, 0)),
            pl.BlockSpec((bn, _D), lambda i: (i + g, 0)),
            pl.BlockSpec((bn,), lambda i: (i,)),
            pl.BlockSpec((bn,), lambda i: (i + g,)),
            pl.BlockSpec((_D, _D), lambda i: (0, 0)),
            pl.BlockSpec((_D, _D), lambda i: (0, 0)),
            pl.BlockSpec((1, _D), lambda i: (0, 0)),
        ],
        out_specs=pl.BlockSpec((bn, _D), lambda i: (i, 0)),
        out_shape=jax.ShapeDtypeStruct((_N_PAD, _D), jnp.float32),
    )(x_pad, parts_flat, parts_flat, degs_flat, degs_flat,
      W_self, W_neigh, b2)


def kernel(x, edge_index, W_self, W_neigh, b):
    src = edge_index[0]
    dst = edge_index[1]
    npad = _E_PAD - _E
    pad_iota = jnp.arange(npad, dtype=jnp.int32)
    src_p = jnp.concatenate([src, pad_iota % _N])
    dst_p = jnp.concatenate([dst, _N + pad_iota % (_N_PAD - _N)])
    eidx = jnp.stack([src_p.reshape(-1, _CH), dst_p.reshape(-1, _CH)], axis=1)
    acc, deg = _sc_gather_segsum(x, eidx)
    x_pad = jnp.concatenate([x, jnp.zeros((_N_PAD - _N, _D), x.dtype)])
    out = _tc_combine(x_pad, acc.reshape(-1, _D), deg.reshape(-1),
                      W_self, W_neigh, b.reshape(1, _D))
    return out[:_N]


# speedup vs baseline: 2.8199x; 1.0775x over previous
import functools

import jax
import jax.numpy as jnp
from jax import lax
from jax.experimental import pallas as pl
from jax.experimental.pallas import tpu as pltpu
from jax.experimental.pallas import tpu_sc as plsc

_N, _E, _D = 10000, 320000, 128
_NC, _NS = 2, 16
_NW = _NC * _NS
_CH = 128
_CPW = 80
_EPW = _CH * _CPW
_E_PAD = _NW * _EPW
_N_PAD = 10240
_RPS = _N_PAD // _NS
_NBUF = 4
_LOOK = 2


def _sc_gather_segsum(x, eidx):
    mesh = plsc.VectorSubcoreMesh(
        core_axis_name="c", subcore_axis_name="s",
        num_cores=_NC, num_subcores=_NS)

    @functools.partial(
        pl.kernel,
        out_type=(jax.ShapeDtypeStruct((_NC, _N_PAD, _D), jnp.float32),
                  jax.ShapeDtypeStruct((_NC, _N_PAD), jnp.float32)),
        mesh=mesh,
        scratch_types=(
            [
                pltpu.VMEM_SHARED((_N_PAD, _D), jnp.float32),
                pltpu.VMEM_SHARED((_N_PAD,), jnp.float32),
                pltpu.VMEM((_CH,), jnp.float32),
            ]
            + [pltpu.VMEM((_CH, _D), jnp.float32)] * 2
            + [pltpu.VMEM((2, _CH), jnp.int32)] * 4
            + [pltpu.SemaphoreType.DMA] * 2
            + [pltpu.SemaphoreType.DMA] * 2
            + [pltpu.SemaphoreType.DMA] * 2
            + [pltpu.SemaphoreType.DMA] * 4
        ))
    def k(x_hbm, e_hbm, acc_out, deg_out, acc_sh, deg_sh, ones_v, *bufs):
        rows = list(bufs[0:2])
        ei = list(bufs[2:6])
        semg = list(bufs[6:8])
        sems = list(bufs[8:10])
        semd = list(bufs[10:12])
        semi = list(bufs[12:16])

        cid = lax.axis_index("c")
        sid = lax.axis_index("s")
        wid = sid * _NC + cid
        cbase = wid * _CPW

        zero16 = jnp.zeros((16,), jnp.float32)
        one16 = jnp.ones((16,), jnp.float32)

        pltpu.async_copy(e_hbm.at[cbase], ei[0], semi[0])
        pltpu.async_copy(e_hbm.at[cbase + 1], ei[1], semi[1])

        def fill_ones(i, carry):
            ones_v[pl.ds(i * 16, 16)] = one16
            return carry
        lax.fori_loop(0, _CH // 16, fill_ones, 0)

        def fill_zero(i, carry):
            rows[0][i // (_D // 16), pl.ds((i % (_D // 16)) * 16, 16)] = zero16
            return carry
        lax.fori_loop(0, _CH * (_D // 16), fill_zero, 0)

        rbase = sid * _RPS

        def zero_acc(j, carry):
            pltpu.sync_copy(rows[0], acc_sh.at[pl.ds(rbase + j * _CH, _CH)])
            pltpu.sync_copy(rows[0].at[0],
                            deg_sh.at[pl.ds(rbase + j * _CH, _CH)])
            return carry
        lax.fori_loop(0, _RPS // _CH, zero_acc, 0)

        plsc.subcore_barrier()

        def wait_rows(sem, buf):
            pltpu.make_async_copy(x_hbm.at[pl.ds(0, _CH)], buf, sem).wait()

        def wait_deg(sem):
            pltpu.make_async_copy(x_hbm.at[0], ones_v, sem).wait()

        def wait_idx(q):
            pltpu.make_async_copy(e_hbm.at[0], ei[q], semi[q]).wait()

        wait_idx(0)
        pltpu.async_copy(x_hbm.at[ei[0].at[0]], rows[0], semg[0])

        def outer(i, carry):
            for b4 in range(4):
                c = i * 4 + b4
                rb = b4 % 2
                ob = 1 - rb
                q = b4
                q1 = (b4 + 1) % 4
                q2 = (b4 + 2) % 4

                def prefetch_idx():
                    pltpu.async_copy(e_hbm.at[cbase + c + 2], ei[q2], semi[q2])
                if b4 < 2:
                    prefetch_idx()
                else:
                    pl.when(i < _CPW // 4 - 1)(prefetch_idx)

                def wait_prev_scatter():
                    wait_rows(sems[ob], rows[ob])
                    wait_deg(semd[ob])
                if b4 > 0:
                    wait_prev_scatter()
                else:
                    pl.when(i > 0)(wait_prev_scatter)

                def gather_next():
                    wait_idx(q1)
                    pltpu.async_copy(x_hbm.at[ei[q1].at[0]], rows[ob],
                                     semg[ob])
                if b4 < 3:
                    gather_next()
                else:
                    pl.when(i < _CPW // 4 - 1)(gather_next)

                wait_rows(semg[rb], rows[rb])
                pltpu.async_copy(rows[rb], acc_sh.at[ei[q].at[1]], sems[rb],
                                 add=True)
                pltpu.async_copy(ones_v, deg_sh.at[ei[q].at[1]], semd[rb],
                                 add=True)
            return carry
        lax.fori_loop(0, _CPW // 4, outer, 0)

        wait_rows(sems[(_CPW - 1) % 2], rows[(_CPW - 1) % 2])
        wait_deg(semd[(_CPW - 1) % 2])

        plsc.subcore_barrier()

        pltpu.sync_copy(acc_sh.at[pl.ds(rbase, _RPS)],
                        acc_out.at[cid, pl.ds(rbase, _RPS)])
        pltpu.sync_copy(deg_sh.at[pl.ds(rbase, _RPS)],
                        deg_out.at[cid, pl.ds(rbase, _RPS)])

    return k(x, eidx)


def _tc_combine(x, parts, degsum, W_self, W_neigh, b2):
    bn = 2048
    g = (_N + bn - 1) // bn

    def body(x_ref, p0_ref, p1_ref, d_ref, ws_ref, wn_ref, b_ref, o_ref):
        deg = jnp.maximum(d_ref[...], 1.0)
        h = (p0_ref[0] + p1_ref[0]) / deg[:, None]
        out = (jnp.dot(x_ref[...], ws_ref[...],
                       preferred_element_type=jnp.float32)
               + jnp.dot(h, wn_ref[...], preferred_element_type=jnp.float32)
               + b_ref[...])
        o_ref[...] = jnp.maximum(out, 0.0)

    return pl.pallas_call(
        body,
        grid=(g,),
        in_specs=[
            pl.BlockSpec((bn, _D), lambda i: (i, 0)),
            pl.BlockSpec((1, bn, _D), lambda i: (0, i, 0)),
            pl.BlockSpec((1, bn, _D), lambda i: (1, i, 0)),
            pl.BlockSpec((bn,), lambda i: (i,)),
            pl.BlockSpec((_D, _D), lambda i: (0, 0)),
            pl.BlockSpec((_D, _D), lambda i: (0, 0)),
            pl.BlockSpec((1, _D), lambda i: (0, 0)),
        ],
        out_specs=pl.BlockSpec((bn, _D), lambda i: (i, 0)),
        out_shape=jax.ShapeDtypeStruct((_N, _D), jnp.float32),
    )(x, parts, parts, degsum, W_self, W_neigh, b2)


def kernel(x, edge_index, W_self, W_neigh, b):
    src = edge_index[0]
    dst = edge_index[1]
    npad = _E_PAD - _E
    pad_iota = jnp.arange(npad, dtype=jnp.int32)
    src_p = jnp.concatenate([src, pad_iota % _N])
    dst_p = jnp.concatenate([dst, _N + pad_iota % (_N_PAD - _N)])
    eidx = jnp.stack([src_p.reshape(-1, _CH), dst_p.reshape(-1, _CH)], axis=1)
    acc, deg = _sc_gather_segsum(x, eidx)
    out = _tc_combine(x, acc, deg[0] + deg[1], W_self, W_neigh,
                      b.reshape(1, _D))
    return out


# speedup vs baseline: 2.9251x; 1.0373x over previous
import functools

import jax
import jax.numpy as jnp
from jax import lax
from jax.experimental import pallas as pl
from jax.experimental.pallas import tpu as pltpu
from jax.experimental.pallas import tpu_sc as plsc

_N, _E, _D = 10000, 320000, 128
_NC, _NS = 2, 16
_NW = _NC * _NS
_CH = 120
_CPW = 84
_EPW = _CH * _CPW
_E_PAD = _NW * _EPW
_N_PAD = 10240
_RPS = _N_PAD // _NS


def _sc_gather_segsum(x, eidx):
    mesh = plsc.VectorSubcoreMesh(
        core_axis_name="c", subcore_axis_name="s",
        num_cores=_NC, num_subcores=_NS)

    @functools.partial(
        pl.kernel,
        out_type=(jax.ShapeDtypeStruct((_NC, _N_PAD, _D), jnp.float32),
                  jax.ShapeDtypeStruct((_NC, _N_PAD), jnp.float32)),
        mesh=mesh,
        scratch_types=(
            [
                pltpu.VMEM_SHARED((_N_PAD, _D), jnp.float32),
                pltpu.VMEM_SHARED((_N_PAD,), jnp.float32),
                pltpu.VMEM((_CH,), jnp.float32),
            ]
            + [pltpu.VMEM((_CH, _D), jnp.float32)] * 3
            + [pltpu.VMEM((2, _CH), jnp.int32)] * 4
            + [pltpu.SemaphoreType.DMA] * 3
            + [pltpu.SemaphoreType.DMA] * 3
            + [pltpu.SemaphoreType.DMA] * 3
            + [pltpu.SemaphoreType.DMA] * 4
        ))
    def k(x_hbm, e_hbm, acc_out, deg_out, acc_sh, deg_sh, ones_v, *bufs):
        rows = list(bufs[0:3])
        ei = list(bufs[3:7])
        semg = list(bufs[7:10])
        sems = list(bufs[10:13])
        semd = list(bufs[13:16])
        semi = list(bufs[16:20])

        cid = lax.axis_index("c")
        sid = lax.axis_index("s")
        wid = sid * _NC + cid
        cbase = wid * _CPW

        zero16 = jnp.zeros((16,), jnp.float32)
        one16 = jnp.ones((16,), jnp.float32)

        pltpu.async_copy(e_hbm.at[cbase], ei[0], semi[0])
        pltpu.async_copy(e_hbm.at[cbase + 1], ei[1], semi[1])

        def fill_ones(i, carry):
            ones_v[pl.ds(i * 16, 16)] = one16
            return carry
        lax.fori_loop(0, 128 // 16, fill_ones, 0)

        def fill_zero(i, carry):
            rows[0][i // (_D // 16), pl.ds((i % (_D // 16)) * 16, 16)] = zero16
            return carry
        lax.fori_loop(0, _CH * (_D // 16), fill_zero, 0)

        rbase = sid * _RPS

        def zero_acc(j, carry):
            pltpu.sync_copy(rows[0], acc_sh.at[pl.ds(rbase + j * _CH, _CH)])
            pltpu.sync_copy(rows[0].at[0, pl.ds(0, _CH)],
                            deg_sh.at[pl.ds(rbase + j * _CH, _CH)])
            return carry
        lax.fori_loop(0, _RPS // _CH, zero_acc, 0)
        tail = _RPS - (_RPS // _CH) * _CH
        if tail:
            pltpu.sync_copy(rows[0].at[pl.ds(0, tail)],
                            acc_sh.at[pl.ds(rbase + _RPS - tail, tail)])
            pltpu.sync_copy(rows[0].at[0, pl.ds(0, tail)],
                            deg_sh.at[pl.ds(rbase + _RPS - tail, tail)])

        plsc.subcore_barrier()

        def wait_rows(sem, buf):
            pltpu.make_async_copy(x_hbm.at[pl.ds(0, _CH)], buf, sem).wait()

        def wait_deg(sem):
            pltpu.make_async_copy(x_hbm.at[0, pl.ds(0, _CH)],
                                  ones_v.at[pl.ds(0, _CH)], sem).wait()

        def wait_idx(q):
            pltpu.make_async_copy(e_hbm.at[0], ei[q], semi[q]).wait()

        wait_idx(0)
        pltpu.async_copy(x_hbm.at[ei[0].at[0]], rows[0], semg[0])

        nI = _CPW // 12

        def outer(i, carry):
            for b12 in range(12):
                c = i * 12 + b12
                r = b12 % 3
                r1 = (b12 + 1) % 3
                q = b12 % 4
                q1 = (b12 + 1) % 4
                q2 = (b12 + 2) % 4

                def wait_prev_scatter():
                    wait_rows(sems[r1], rows[r1])
                    wait_deg(semd[r1])
                if b12 >= 2:
                    wait_prev_scatter()
                else:
                    pl.when(i > 0)(wait_prev_scatter)

                def prefetch_idx():
                    pltpu.async_copy(e_hbm.at[cbase + c + 2], ei[q2], semi[q2])
                if b12 < 10:
                    prefetch_idx()
                else:
                    pl.when(i < nI - 1)(prefetch_idx)

                def gather_next():
                    wait_idx(q1)
                    pltpu.async_copy(x_hbm.at[ei[q1].at[0]], rows[r1],
                                     semg[r1])
                if b12 < 11:
                    gather_next()
                else:
                    pl.when(i < nI - 1)(gather_next)

                wait_rows(semg[r], rows[r])
                pltpu.async_copy(rows[r], acc_sh.at[ei[q].at[1]], sems[r],
                                 add=True)
                pltpu.async_copy(ones_v.at[pl.ds(0, _CH)],
                                 deg_sh.at[ei[q].at[1]], semd[r], add=True)
            return carry
        lax.fori_loop(0, nI, outer, 0)

        for cc in (_CPW - 2, _CPW - 1):
            wait_rows(sems[cc % 3], rows[cc % 3])
            wait_deg(semd[cc % 3])

        plsc.subcore_barrier()

        pltpu.sync_copy(acc_sh.at[pl.ds(rbase, _RPS)],
                        acc_out.at[cid, pl.ds(rbase, _RPS)])
        pltpu.sync_copy(deg_sh.at[pl.ds(rbase, _RPS)],
                        deg_out.at[cid, pl.ds(rbase, _RPS)])

    return k(x, eidx)


def _tc_combine(x, parts, degsum, W_self, W_neigh, b2):
    bn = 2048
    g = (_N + bn - 1) // bn

    def body(x_ref, p0_ref, p1_ref, d_ref, ws_ref, wn_ref, b_ref, o_ref):
        deg = jnp.maximum(d_ref[...], 1.0)
        h = (p0_ref[0] + p1_ref[0]) / deg[:, None]
        out = (jnp.dot(x_ref[...], ws_ref[...],
                       preferred_element_type=jnp.float32)
               + jnp.dot(h, wn_ref[...], preferred_element_type=jnp.float32)
               + b_ref[...])
        o_ref[...] = jnp.maximum(out, 0.0)

    return pl.pallas_call(
        body,
        grid=(g,),
        in_specs=[
            pl.BlockSpec((bn, _D), lambda i: (i, 0)),
            pl.BlockSpec((1, bn, _D), lambda i: (0, i, 0)),
            pl.BlockSpec((1, bn, _D), lambda i: (1, i, 0)),
            pl.BlockSpec((bn,), lambda i: (i,)),
            pl.BlockSpec((_D, _D), lambda i: (0, 0)),
            pl.BlockSpec((_D, _D), lambda i: (0, 0)),
            pl.BlockSpec((1, _D), lambda i: (0, 0)),
        ],
        out_specs=pl.BlockSpec((bn, _D), lambda i: (i, 0)),
        out_shape=jax.ShapeDtypeStruct((_N, _D), jnp.float32),
    )(x, parts, parts, degsum, W_self, W_neigh, b2)


def kernel(x, edge_index, W_self, W_neigh, b):
    src = edge_index[0]
    dst = edge_index[1]
    npad = _E_PAD - _E
    pad_iota = jnp.arange(npad, dtype=jnp.int32)
    src_p = jnp.concatenate([src, pad_iota % _N])
    dst_p = jnp.concatenate([dst, _N + pad_iota % (_N_PAD - _N)])
    eidx = jnp.stack([src_p.reshape(-1, _CH), dst_p.reshape(-1, _CH)], axis=1)
    acc, deg = _sc_gather_segsum(x, eidx)
    out = _tc_combine(x, acc, deg[0] + deg[1], W_self, W_neigh,
                      b.reshape(1, _D))
    return out


# speedup vs baseline: 2.9292x; 1.0014x over previous
import functools

import jax
import jax.numpy as jnp
from jax import lax
from jax.experimental import pallas as pl
from jax.experimental.pallas import tpu as pltpu
from jax.experimental.pallas import tpu_sc as plsc

_N, _E, _D = 10000, 320000, 128
_NC, _NS = 2, 16
_NW = _NC * _NS
_CH = 120
_CPW = 84
_EPW = _CH * _CPW
_E_PAD = _NW * _EPW
_N_PAD = 10240
_RPS = _N_PAD // _NS


def _sc_gather_segsum(x, eidx):
    mesh = plsc.VectorSubcoreMesh(
        core_axis_name="c", subcore_axis_name="s",
        num_cores=_NC, num_subcores=_NS)

    @functools.partial(
        pl.kernel,
        out_type=(jax.ShapeDtypeStruct((_NC, _N_PAD, _D), jnp.float32),
                  jax.ShapeDtypeStruct((_NC, _N_PAD), jnp.float32)),
        mesh=mesh,
        scratch_types=(
            [
                pltpu.VMEM_SHARED((_N_PAD, _D), jnp.float32),
                pltpu.VMEM_SHARED((_N_PAD,), jnp.float32),
                pltpu.VMEM((_CH,), jnp.float32),
            ]
            + [pltpu.VMEM((_CH, _D), jnp.float32)] * 3
            + [pltpu.VMEM((2, _CH), jnp.int32)] * 4
            + [pltpu.SemaphoreType.DMA] * 3
            + [pltpu.SemaphoreType.DMA] * 3
            + [pltpu.SemaphoreType.DMA] * 3
            + [pltpu.SemaphoreType.DMA] * 4
        ))
    def k(x_hbm, e_hbm, acc_out, deg_out, acc_sh, deg_sh, ones_v, *bufs):
        rows = list(bufs[0:3])
        ei = list(bufs[3:7])
        semg = list(bufs[7:10])
        sems = list(bufs[10:13])
        semd = list(bufs[13:16])
        semi = list(bufs[16:20])

        cid = lax.axis_index("c")
        sid = lax.axis_index("s")
        wid = sid * _NC + cid
        cbase = wid * _CPW

        zero16 = jnp.zeros((16,), jnp.float32)
        one16 = jnp.ones((16,), jnp.float32)

        pltpu.async_copy(e_hbm.at[cbase], ei[0], semi[0])
        pltpu.async_copy(e_hbm.at[cbase + 1], ei[1], semi[1])

        def fill_ones(i, carry):
            ones_v[pl.ds(i * 16, 16)] = one16
            return carry
        lax.fori_loop(0, 128 // 16, fill_ones, 0)

        def fill_zero(i, carry):
            rows[0][i // (_D // 16), pl.ds((i % (_D // 16)) * 16, 16)] = zero16
            return carry
        lax.fori_loop(0, _CH * (_D // 16), fill_zero, 0)

        rbase = sid * _RPS

        def zero_acc(j, carry):
            pltpu.sync_copy(rows[0], acc_sh.at[pl.ds(rbase + j * _CH, _CH)])
            pltpu.sync_copy(rows[0].at[0, pl.ds(0, _CH)],
                            deg_sh.at[pl.ds(rbase + j * _CH, _CH)])
            return carry
        lax.fori_loop(0, _RPS // _CH, zero_acc, 0)
        tail = _RPS - (_RPS // _CH) * _CH
        if tail:
            pltpu.sync_copy(rows[0].at[pl.ds(0, tail)],
                            acc_sh.at[pl.ds(rbase + _RPS - tail, tail)])
            pltpu.sync_copy(rows[0].at[0, pl.ds(0, tail)],
                            deg_sh.at[pl.ds(rbase + _RPS - tail, tail)])

        plsc.subcore_barrier()

        def wait_rows(sem, buf):
            pltpu.make_async_copy(x_hbm.at[pl.ds(0, _CH)], buf, sem).wait()

        def wait_deg(sem):
            pltpu.make_async_copy(x_hbm.at[0, pl.ds(0, _CH)],
                                  ones_v.at[pl.ds(0, _CH)], sem).wait()

        def wait_idx(q):
            pltpu.make_async_copy(e_hbm.at[0], ei[q], semi[q]).wait()

        wait_idx(0)
        pltpu.async_copy(x_hbm.at[ei[0].at[0]], rows[0], semg[0])

        nI = _CPW // 12

        def outer(i, carry):
            for b12 in range(12):
                c = i * 12 + b12
                r = b12 % 3
                r1 = (b12 + 1) % 3
                q = b12 % 4
                q1 = (b12 + 1) % 4
                q2 = (b12 + 2) % 4

                def wait_prev_scatter():
                    wait_rows(sems[r1], rows[r1])
                    wait_deg(semd[r1])
                if b12 >= 2:
                    wait_prev_scatter()
                else:
                    pl.when(i > 0)(wait_prev_scatter)

                def prefetch_idx():
                    pltpu.async_copy(e_hbm.at[cbase + c + 2], ei[q2], semi[q2])
                if b12 < 10:
                    prefetch_idx()
                else:
                    pl.when(i < nI - 1)(prefetch_idx)

                def gather_next():
                    wait_idx(q1)
                    pltpu.async_copy(x_hbm.at[ei[q1].at[0]], rows[r1],
                                     semg[r1])
                if b12 < 11:
                    gather_next()
                else:
                    pl.when(i < nI - 1)(gather_next)

                wait_rows(semg[r], rows[r])
                pltpu.async_copy(rows[r], acc_sh.at[ei[q].at[1]], sems[r],
                                 add=True)
                pltpu.async_copy(ones_v.at[pl.ds(0, _CH)],
                                 deg_sh.at[ei[q].at[1]], semd[r], add=True)
            return carry
        lax.fori_loop(0, nI, outer, 0)

        for cc in (_CPW - 2, _CPW - 1):
            wait_rows(sems[cc % 3], rows[cc % 3])
            wait_deg(semd[cc % 3])

        plsc.subcore_barrier()

        pltpu.sync_copy(acc_sh.at[pl.ds(rbase, _RPS)],
                        acc_out.at[cid, pl.ds(rbase, _RPS)])
        pltpu.sync_copy(deg_sh.at[pl.ds(rbase, _RPS)],
                        deg_out.at[cid, pl.ds(rbase, _RPS)])

    return k(x, eidx)


def _tc_self(x, W_self, b2):
    bn = 2048
    g = (_N + bn - 1) // bn

    def body(x_ref, ws_ref, b_ref, o_ref):
        o_ref[...] = jnp.dot(x_ref[...], ws_ref[...],
                             preferred_element_type=jnp.float32) + b_ref[...]

    return pl.pallas_call(
        body,
        grid=(g,),
        in_specs=[
            pl.BlockSpec((bn, _D), lambda i: (i, 0)),
            pl.BlockSpec((_D, _D), lambda i: (0, 0)),
            pl.BlockSpec((1, _D), lambda i: (0, 0)),
        ],
        out_specs=pl.BlockSpec((bn, _D), lambda i: (i, 0)),
        out_shape=jax.ShapeDtypeStruct((_N, _D), jnp.float32),
    )(x, W_self, b2)


def _tc_combine(s, parts, degsum, W_neigh):
    bn = 2048
    g = (_N + bn - 1) // bn

    def body(s_ref, p0_ref, p1_ref, d_ref, wn_ref, o_ref):
        deg = jnp.maximum(d_ref[...], 1.0)
        h = (p0_ref[0] + p1_ref[0]) / deg[:, None]
        out = s_ref[...] + jnp.dot(h, wn_ref[...],
                                   preferred_element_type=jnp.float32)
        o_ref[...] = jnp.maximum(out, 0.0)

    return pl.pallas_call(
        body,
        grid=(g,),
        in_specs=[
            pl.BlockSpec((bn, _D), lambda i: (i, 0)),
            pl.BlockSpec((1, bn, _D), lambda i: (0, i, 0)),
            pl.BlockSpec((1, bn, _D), lambda i: (1, i, 0)),
            pl.BlockSpec((bn,), lambda i: (i,)),
            pl.BlockSpec((_D, _D), lambda i: (0, 0)),
        ],
        out_specs=pl.BlockSpec((bn, _D), lambda i: (i, 0)),
        out_shape=jax.ShapeDtypeStruct((_N, _D), jnp.float32),
    )(s, parts, parts, degsum, W_neigh)


def kernel(x, edge_index, W_self, W_neigh, b):
    src = edge_index[0]
    dst = edge_index[1]
    npad = _E_PAD - _E
    pad_iota = jnp.arange(npad, dtype=jnp.int32)
    src_p = jnp.concatenate([src, pad_iota % _N])
    dst_p = jnp.concatenate([dst, _N + pad_iota % (_N_PAD - _N)])
    eidx = jnp.stack([src_p.reshape(-1, _CH), dst_p.reshape(-1, _CH)], axis=1)
    acc, deg = _sc_gather_segsum(x, eidx)
    s = _tc_self(x, W_self, b.reshape(1, _D))
    return _tc_combine(s, acc, deg[0] + deg[1], W_neigh)
